# baseline (speedup 1.0000x reference)
import jax
import jax.numpy as jnp
from jax import lax
from jax.experimental import pallas as pl
from jax.experimental.pallas import tpu as pltpu, tpu_sc as plsc

_B = 16
_N = 65536
_NI = 16384
_P = 3
_HN = _N // 2
_OQ = _NI // 4
_UNROLL = 16


def _gather_quarter(idx_v, ring_v, dst, q, lo_third, hi_third):
    off_lo = jnp.int32(lo_third * _HN)
    off_hi = jnp.int32(hi_third * _HN)

    def chunk_body(k, carry):
        vals = []
        for u in range(_UNROLL):
            o = (k * _UNROLL + u) * 16
            v = idx_v[pl.ds(q * _OQ + o, 16)]
            vm = v & jnp.int32(_HN - 1)
            base = jnp.where((v >> 15) != 0, off_hi, off_lo)
            vals.append(plsc.load_gather(ring_v, [vm + base]))
        for u in range(_UNROLL):
            o = (k * _UNROLL + u) * 16
            dst[pl.ds(o, 16)] = vals[u]
        return carry

    lax.fori_loop(0, _OQ // (16 * _UNROLL), chunk_body, 0)


def _gather_plane(idx_v, ring_v, outq0, outq1, out_row, lo_third, hi_third, so):
    handles = [None, None]
    for q in range(4):
        dst = outq0 if q % 2 == 0 else outq1
        if handles[q % 2] is not None:
            handles[q % 2].wait()
        _gather_quarter(idx_v, ring_v, dst, q, lo_third, hi_third)
        handles[q % 2] = pltpu.async_copy(
            dst, out_row.at[pl.ds(q * _OQ, _OQ)], so
        )
    return handles


def _gather_body(t_t_hbm, t_idx_hbm, out_hbm, ring_v, idx_v, outq0, outq1,
                 si, s0, s1, s2, so):
    wid = lax.axis_index("s") * 2 + lax.axis_index("c")
    is_a = wid < _B
    b = jnp.where(is_a, wid, wid - _B)
    c0 = jnp.where(is_a, 0, 2)
    sems = (s0, s1, s2)

    def stage(c, hi, third):
        return pltpu.async_copy(
            t_t_hbm.at[c, b, pl.ds(hi * _HN, _HN)],
            ring_v.at[pl.ds(third * _HN, _HN)],
            sems[third],
        )

    hidx = pltpu.async_copy(t_idx_hbm.at[b], idx_v, si)
    h_lo0 = stage(c0, 0, 0)
    h_hi0 = stage(c0, 1, 1)
    hidx.wait()
    h_lo0.wait()
    h_hi0.wait()

    @pl.when(is_a)
    def _():
        h_lo1 = stage(1, 0, 2)
        hs = _gather_plane(idx_v, ring_v, outq0, outq1, out_hbm.at[0, b], 0, 1, so)
        h_hi1 = stage(1, 1, 0)
        hs[0].wait()
        hs[1].wait()
        h_lo1.wait()
        h_hi1.wait()
        hs2 = _gather_plane(idx_v, ring_v, outq0, outq1, out_hbm.at[1, b], 2, 0, so)
        hs2[0].wait()
        hs2[1].wait()

    @pl.when(jnp.logical_not(is_a))
    def _():
        hs = _gather_plane(idx_v, ring_v, outq0, outq1, out_hbm.at[2, b], 0, 1, so)
        hs[0].wait()
        hs[1].wait()


def kernel(t_in, t_idx):
    b, n, p = t_in.shape
    nidx = t_idx.shape[1]
    t_t = jnp.transpose(t_in, (2, 0, 1))
    idx = t_idx.astype(jnp.int32)
    mesh = plsc.VectorSubcoreMesh(core_axis_name="c", subcore_axis_name="s")
    out = pl.kernel(
        _gather_body,
        out_type=jax.ShapeDtypeStruct((p, b, nidx), jnp.float32),
        mesh=mesh,
        compiler_params=pltpu.CompilerParams(
            use_tc_tiling_on_sc=True, needs_layout_passes=False
        ),
        scratch_types=[
            pltpu.VMEM((3 * _HN,), jnp.float32),
            pltpu.VMEM((_NI,), jnp.int32),
            pltpu.VMEM((_OQ,), jnp.float32),
            pltpu.VMEM((_OQ,), jnp.float32),
            pltpu.SemaphoreType.DMA,
            pltpu.SemaphoreType.DMA,
            pltpu.SemaphoreType.DMA,
            pltpu.SemaphoreType.DMA,
            pltpu.SemaphoreType.DMA,
        ],
    )(t_t, idx)
    return jnp.transpose(out, (1, 2, 0))


# speedup vs baseline: 1.0132x; 1.0132x over previous
import jax
import jax.numpy as jnp
from jax import lax
from jax.experimental import pallas as pl
from jax.experimental.pallas import tpu as pltpu, tpu_sc as plsc

_B = 16
_N = 65536
_NI = 16384
_P = 3
_UNROLL = 16


def _gather_range(idx_v, plane_v, dst, ioff, count):
    def chunk_body(k, carry):
        vals = []
        for u in range(_UNROLL):
            o = (k * _UNROLL + u) * 16
            v = idx_v[pl.ds(ioff + o, 16)]
            vals.append(plsc.load_gather(plane_v, [v]))
        for u in range(_UNROLL):
            o = (k * _UNROLL + u) * 16
            dst[pl.ds(o, 16)] = vals[u]
        return carry

    lax.fori_loop(0, count // (16 * _UNROLL), chunk_body, 0)


def _gather_body(t_t_hbm, t_idx_hbm, out_hbm, plane_v, idx_v, outv0, outv1,
                 si, sp, so):
    wid = lax.axis_index("s") * 2 + lax.axis_index("c")
    is_a = wid < _B
    b = jnp.where(is_a, wid, wid - _B)
    c0 = jnp.where(is_a, 0, 2)

    hidx = pltpu.async_copy(t_idx_hbm.at[b], idx_v, si)
    hplane = pltpu.async_copy(t_t_hbm.at[c0, b], plane_v, sp)
    hidx.wait()
    hplane.wait()
    _gather_range(idx_v, plane_v, outv0, 0, _NI)
    o0 = pltpu.async_copy(outv0, out_hbm.at[c0, b], so)

    hoff = jnp.where(is_a, 0, _NI // 2)
    pltpu.sync_copy(t_t_hbm.at[1, b], plane_v)
    _gather_range(idx_v, plane_v, outv1, hoff, _NI // 2)
    pltpu.sync_copy(
        outv1.at[pl.ds(0, _NI // 2)], out_hbm.at[1, b, pl.ds(hoff, _NI // 2)]
    )

    o0.wait()


def kernel(t_in, t_idx):
    b, n, p = t_in.shape
    nidx = t_idx.shape[1]
    t_t = jnp.transpose(t_in, (2, 0, 1))
    idx = t_idx.astype(jnp.int32)
    mesh = plsc.VectorSubcoreMesh(core_axis_name="c", subcore_axis_name="s")
    out = pl.kernel(
        _gather_body,
        out_type=jax.ShapeDtypeStruct((p, b, nidx), jnp.float32),
        mesh=mesh,
        compiler_params=pltpu.CompilerParams(
            use_tc_tiling_on_sc=True, needs_layout_passes=False
        ),
        scratch_types=[
            pltpu.VMEM((_N,), jnp.float32),
            pltpu.VMEM((_NI,), jnp.int32),
            pltpu.VMEM((_NI,), jnp.float32),
            pltpu.VMEM((_NI,), jnp.float32),
            pltpu.SemaphoreType.DMA,
            pltpu.SemaphoreType.DMA,
            pltpu.SemaphoreType.DMA,
        ],
    )(t_t, idx)
    return jnp.transpose(out, (1, 2, 0))


# speedup vs baseline: 1.0174x; 1.0041x over previous
import jax
import jax.numpy as jnp
from jax import lax
from jax.experimental import pallas as pl
from jax.experimental.pallas import tpu as pltpu, tpu_sc as plsc

_B = 16
_N = 65536
_NI = 16384
_P = 3
_UNROLL = 16


def _gather_all(idx_v, plane_v, dst):
    def chunk_body(k, carry):
        vals = []
        for u in range(_UNROLL):
            o = (k * _UNROLL + u) * 16
            v = idx_v[pl.ds(o, 16)]
            vals.append(plsc.load_gather(plane_v, [v]))
        for u in range(_UNROLL):
            o = (k * _UNROLL + u) * 16
            dst[pl.ds(o, 16)] = vals[u]
        return carry

    lax.fori_loop(0, _NI // (16 * _UNROLL), chunk_body, 0)


def _gather_body(t_t_hbm, t_idx_hbm, out_hbm, plane_v, idx_v, outv0, outv1,
                 si, sp, so):
    wid = lax.axis_index("s") * 2 + lax.axis_index("c")
    is_a = wid < _B
    b = jnp.where(is_a, wid, wid - _B)
    c0 = jnp.where(is_a, 0, 2)

    hidx = pltpu.async_copy(t_idx_hbm.at[b], idx_v, si)
    hplane = pltpu.async_copy(t_t_hbm.at[c0, b], plane_v, sp)
    hidx.wait()
    hplane.wait()
    _gather_all(idx_v, plane_v, outv0)
    o0 = pltpu.async_copy(outv0, out_hbm.at[c0, b], so)

    @pl.when(is_a)
    def _():
        pltpu.sync_copy(t_t_hbm.at[1, b], plane_v)
        _gather_all(idx_v, plane_v, outv1)
        pltpu.sync_copy(outv1, out_hbm.at[1, b])

    o0.wait()


def kernel(t_in, t_idx):
    b, n, p = t_in.shape
    nidx = t_idx.shape[1]
    t_t = jnp.transpose(t_in, (2, 0, 1))
    idx = t_idx.astype(jnp.int32)
    mesh = plsc.VectorSubcoreMesh(core_axis_name="c", subcore_axis_name="s")
    out = pl.kernel(
        _gather_body,
        out_type=jax.ShapeDtypeStruct((p, b, nidx), jnp.float32),
        mesh=mesh,
        compiler_params=pltpu.CompilerParams(
            use_tc_tiling_on_sc=True, needs_layout_passes=False
        ),
        scratch_types=[
            pltpu.VMEM((_N,), jnp.float32),
            pltpu.VMEM((_NI,), jnp.int32),
            pltpu.VMEM((_NI,), jnp.float32),
            pltpu.VMEM((_NI,), jnp.float32),
            pltpu.SemaphoreType.DMA,
            pltpu.SemaphoreType.DMA,
            pltpu.SemaphoreType.DMA,
        ],
    )(t_t, idx)
    return jnp.transpose(out, (1, 2, 0))
